# baseline (speedup 1.0000x reference)
import functools

import jax
import jax.numpy as jnp
import numpy as np
from jax import lax
from jax.experimental import pallas as pl
from jax.experimental.pallas import tpu as pltpu
from jax.experimental.pallas import tpu_sc as plsc

F = 26
E = 16
B = 16384
ROWS_PER_FIELD = 38462
TOTAL_ROWS = F * ROWS_PER_FIELD
_OFFSETS = np.arange(F, dtype=np.int32) * ROWS_PER_FIELD

NW = 32
FLAT = B * F
PER_W = FLAT // NW
IW = 128
NROW = PER_W // IW
BW = B // NW

_OFF_PAT = _OFFSETS[(np.arange(PER_W) % F)].reshape(NROW, IW)


def _sc_body(xf, xt, emb_t, fc_f, off, bias16, emb_out, lr_out,
             x_v, gid_v, off_v, xt_v, gidf_v, ebuf, fcv, lr_v, bias_v):
    wid = lax.axis_index("s") * 2 + lax.axis_index("c")
    base = wid * PER_W

    pltpu.sync_copy(xf.at[pl.ds(wid * NROW, NROW)], x_v)
    pltpu.sync_copy(off, off_v)
    pltpu.sync_copy(bias16, bias_v)
    pltpu.sync_copy(xt.at[:, pl.ds(wid * BW, BW)], xt_v)

    @pl.loop(0, NROW)
    def _(r):
        for c in range(IW // 16):
            sl = pl.ds(c * 16, 16)
            gid_v[r, sl] = x_v[r, sl] + off_v[r, sl]

    @pl.loop(0, F)
    def _(f):
        o = f * ROWS_PER_FIELD
        for q in range(BW // IW):
            for c in range(IW // 16):
                gidf_v[f * (BW // IW) + q, pl.ds(c * 16, 16)] = (
                    xt_v[f, pl.ds(q * IW + c * 16, 16)] + o)

    @pl.loop(0, NROW)
    def _(j):
        pltpu.sync_copy(emb_t.at[gid_v.at[j]], ebuf)
        pltpu.sync_copy(ebuf, emb_out.at[pl.ds(base + j * IW, IW)])

    @pl.loop(0, NROW)
    def _(j):
        pltpu.sync_copy(fc_f.at[gidf_v.at[j]], fcv.at[pl.ds(j * IW, IW)])

    @pl.loop(0, BW // 16)
    def _(g):
        acc = bias_v[...]
        for f in range(F):
            acc = acc + fcv[pl.ds(f * BW + g * 16, 16)]
        lr_v[pl.ds(g * 16, 16)] = acc

    pltpu.sync_copy(lr_v, lr_out.at[pl.ds(wid * BW, BW)])


@jax.jit
def _sc_call(xf, xt, emb_t, fc_f, off, bias16):
    mesh = plsc.VectorSubcoreMesh(core_axis_name="c", subcore_axis_name="s")
    return pl.kernel(
        _sc_body,
        compiler_params=pltpu.CompilerParams(use_tc_tiling_on_sc=False),
        out_type=(
            jax.ShapeDtypeStruct((FLAT, E), jnp.float32),
            jax.ShapeDtypeStruct((B,), jnp.float32),
        ),
        mesh=mesh,
        scratch_types=[
            pltpu.VMEM((NROW, IW), jnp.int32),
            pltpu.VMEM((NROW, IW), jnp.int32),
            pltpu.VMEM((NROW, IW), jnp.int32),
            pltpu.VMEM((F, BW), jnp.int32),
            pltpu.VMEM((NROW, IW), jnp.int32),
            pltpu.VMEM((IW, E), jnp.float32),
            pltpu.VMEM((PER_W,), jnp.float32),
            pltpu.VMEM((BW,), jnp.float32),
            pltpu.VMEM((16,), jnp.float32),
        ],
    )(xf, xt, emb_t, fc_f, off, bias16)


def kernel(x, emb_table, fc_table, bias):
    xf = x.reshape(FLAT // IW, IW)
    xt = x.T
    fc_f = fc_table.reshape(TOTAL_ROWS)
    bias16 = jnp.broadcast_to(bias.astype(jnp.float32), (16,))
    emb_flat, lr = _sc_call(xf, xt, emb_table, fc_f, jnp.asarray(_OFF_PAT),
                            bias16)
    return emb_flat.reshape(B, F, E), lr.reshape(B, 1)


# speedup vs baseline: 1.1283x; 1.1283x over previous
import jax
import jax.numpy as jnp
import numpy as np
from jax import lax
from jax.experimental import pallas as pl
from jax.experimental.pallas import tpu as pltpu
from jax.experimental.pallas import tpu_sc as plsc

F = 26
E = 16
B = 16384
ROWS_PER_FIELD = 38462
TOTAL_ROWS = F * ROWS_PER_FIELD
_OFFSETS = np.arange(F, dtype=np.int32) * ROWS_PER_FIELD

NW = 32
FLAT = B * F
PER_W = FLAT // NW
IW = 128
NROW = PER_W // IW
BW = B // NW

SLOTS = 8
LEAD = 4
MAIN = 96

_OFF_PAT = _OFFSETS[(np.arange(PER_W) % F)].reshape(NROW, IW)


def _sc_body(xf, xt, emb_t, fc_f, off, bias16, emb_out, lr_out,
             x_v, gid_v, off_v, xt_v, gidf_v, ebig, fcv, lr_v, bias_v,
             gsem, wsem, fcsem, ldsem):
    wid = lax.axis_index("s") * 2 + lax.axis_index("c")
    base = wid * PER_W

    c1 = pltpu.async_copy(xf.at[pl.ds(wid * NROW, NROW)], x_v, ldsem)
    c2 = pltpu.async_copy(off, off_v, ldsem)
    c3 = pltpu.async_copy(bias16, bias_v, ldsem)
    c4 = pltpu.async_copy(xt.at[:, pl.ds(wid * BW, BW)], xt_v, ldsem)
    c1.wait(); c2.wait(); c3.wait(); c4.wait()

    @pl.loop(0, NROW)
    def _(r):
        for c in range(IW // 16):
            sl = pl.ds(c * 16, 16)
            gid_v[r, sl] = x_v[r, sl] + off_v[r, sl]

    @pl.loop(0, F)
    def _(f):
        o = f * ROWS_PER_FIELD
        for q in range(BW // IW):
            for c in range(IW // 16):
                gidf_v[f * (BW // IW) + q, pl.ds(c * 16, 16)] = (
                    xt_v[f, pl.ds(q * IW + c * 16, 16)] + o)

    def g_desc(j, s):
        return pltpu.make_async_copy(emb_t.at[gid_v.at[j]],
                                     ebig.at[pl.ds(s * IW, IW)], gsem.at[s])

    def w_desc(j, s):
        return pltpu.make_async_copy(ebig.at[pl.ds(s * IW, IW)],
                                     emb_out.at[pl.ds(base + j * IW, IW)],
                                     wsem.at[s])

    def fc_desc(j):
        return pltpu.make_async_copy(fc_f.at[gidf_v.at[j]],
                                     fcv.at[pl.ds(j * IW, IW)], fcsem)

    def step(j, t, fire_next):
        g_desc(j, t).wait()
        fc_desc(j).start()

        @pl.when(j >= LEAD)
        def _():
            fc_desc(j - LEAD).wait()

        w_desc(j, t).start()
        if fire_next:
            s2 = (t + LEAD) % SLOTS
            j2 = j + LEAD

            @pl.when(j2 >= SLOTS)
            def _():
                w_desc(j2 - SLOTS, s2).wait()

            g_desc(j2, s2).start()

    for s in range(LEAD):
        g_desc(s, s).start()

    @pl.loop(0, MAIN // SLOTS)
    def _(g):
        for t in range(SLOTS):
            step(g * SLOTS + t, t, fire_next=True)

    for j in range(MAIN, NROW):
        step(j, j % SLOTS, fire_next=(j + LEAD < NROW))

    for j in range(NROW - LEAD, NROW):
        fc_desc(j).wait()

    @pl.loop(0, BW // 16)
    def _(g):
        acc = bias_v[...]
        for f in range(F):
            acc = acc + fcv[pl.ds(f * BW + g * 16, 16)]
        lr_v[pl.ds(g * 16, 16)] = acc

    pltpu.sync_copy(lr_v, lr_out.at[pl.ds(wid * BW, BW)])

    for j in range(NROW - SLOTS, NROW):
        w_desc(j, j % SLOTS).wait()


@jax.jit
def _sc_call(xf, xt, emb_t, fc_f, off, bias16):
    mesh = plsc.VectorSubcoreMesh(core_axis_name="c", subcore_axis_name="s")
    return pl.kernel(
        _sc_body,
        out_type=(
            jax.ShapeDtypeStruct((FLAT, E), jnp.float32),
            jax.ShapeDtypeStruct((B,), jnp.float32),
        ),
        mesh=mesh,
        compiler_params=pltpu.CompilerParams(use_tc_tiling_on_sc=False),
        scratch_types=[
            pltpu.VMEM((NROW, IW), jnp.int32),
            pltpu.VMEM((NROW, IW), jnp.int32),
            pltpu.VMEM((NROW, IW), jnp.int32),
            pltpu.VMEM((F, BW), jnp.int32),
            pltpu.VMEM((NROW, IW), jnp.int32),
            pltpu.VMEM((SLOTS * IW, E), jnp.float32),
            pltpu.VMEM((PER_W,), jnp.float32),
            pltpu.VMEM((BW,), jnp.float32),
            pltpu.VMEM((16,), jnp.float32),
            pltpu.SemaphoreType.DMA((SLOTS,)),
            pltpu.SemaphoreType.DMA((SLOTS,)),
            pltpu.SemaphoreType.DMA,
            pltpu.SemaphoreType.DMA,
        ],
    )(xf, xt, emb_t, fc_f, off, bias16)


def kernel(x, emb_table, fc_table, bias):
    xf = x.reshape(FLAT // IW, IW)
    xt = x.T
    fc_f = fc_table.reshape(TOTAL_ROWS)
    bias16 = jnp.broadcast_to(bias.astype(jnp.float32), (16,))
    emb_flat, lr = _sc_call(xf, xt, emb_table, fc_f, jnp.asarray(_OFF_PAT),
                            bias16)
    return emb_flat.reshape(B, F, E), lr.reshape(B, 1)
